# Initial kernel scaffold; baseline (speedup 1.0000x reference)
import functools

import jax
import jax.numpy as jnp
from jax import lax
from jax.experimental import pallas as pl
from jax.experimental.pallas import tpu as pltpu
from jax.experimental.pallas import tpu_sc as plsc

NC = 2
NS = 16
NW = NC * NS

EMB_DIM = 32
CHUNK = 128


def _body(idx_hbm, table_hbm, out_hbm, idx_v, rows_v, gsem):
    nchunk = idx_hbm.shape[1]
    wid = lax.axis_index("s") * NC + lax.axis_index("c")
    base = wid * (nchunk * CHUNK)
    pltpu.sync_copy(idx_hbm.at[wid], idx_v)

    @pl.loop(0, nchunk)
    def _(j):
        pltpu.async_copy(table_hbm.at[idx_v.at[j]], rows_v, gsem).wait()
        pltpu.sync_copy(rows_v, out_hbm.at[pl.ds(base + j * CHUNK, CHUNK)])


def kernel(indices, emb_table):
    b, w = indices.shape
    n = b * w
    nchunk = n // (NW * CHUNK)
    idx3 = indices.astype(jnp.int32).reshape(NW, nchunk, CHUNK)

    run = pl.kernel(
        _body,
        out_type=jax.ShapeDtypeStruct((n, EMB_DIM), jnp.float32),
        mesh=plsc.VectorSubcoreMesh(
            core_axis_name="c", subcore_axis_name="s",
            num_cores=NC, num_subcores=NS),
        scratch_types=[
            pltpu.VMEM((nchunk, CHUNK), jnp.int32),
            pltpu.VMEM((CHUNK, EMB_DIM), jnp.float32),
            pltpu.SemaphoreType.DMA,
        ],
    )
    out = run(idx3, emb_table)
    return out.reshape(b, w, EMB_DIM)


# speedup vs baseline: 1.4021x; 1.4021x over previous
import functools

import jax
import jax.numpy as jnp
from jax import lax
from jax.experimental import pallas as pl
from jax.experimental.pallas import tpu as pltpu
from jax.experimental.pallas import tpu_sc as plsc

NC = 2
NS = 16
NW = NC * NS

EMB_DIM = 32
CHUNK = 128


def _body(idx_hbm, table_hbm, out_hbm, idx_v, rows_v, gsem):
    nchunk = idx_hbm.shape[1]
    wid = lax.axis_index("s") * NC + lax.axis_index("c")
    base = wid * (nchunk * CHUNK)
    pltpu.sync_copy(idx_hbm.at[wid], idx_v)

    @pl.loop(0, nchunk)
    def _(j):
        pltpu.async_copy(table_hbm.at[idx_v.at[j]], rows_v, gsem).wait()
        pltpu.sync_copy(rows_v, out_hbm.at[pl.ds(base + j * CHUNK, CHUNK)])


def kernel(indices, emb_table):
    b, w = indices.shape
    n = b * w
    nchunk = n // (NW * CHUNK)
    idx3 = indices.astype(jnp.int32).reshape(NW, nchunk, CHUNK)

    run = pl.kernel(
        _body,
        out_type=jax.ShapeDtypeStruct((n, EMB_DIM), jnp.float32),
        mesh=plsc.VectorSubcoreMesh(
            core_axis_name="c", subcore_axis_name="s",
            num_cores=NC, num_subcores=NS),
        scratch_types=[
            pltpu.VMEM((nchunk, CHUNK), jnp.int32),
            pltpu.VMEM((CHUNK, EMB_DIM), jnp.float32),
            pltpu.SemaphoreType.DMA,
        ],
        compiler_params=pltpu.CompilerParams(use_tc_tiling_on_sc=False),
    )
    out = run(idx3, emb_table)
    return out.reshape(b, w, EMB_DIM)


# speedup vs baseline: 1.5069x; 1.0748x over previous
import jax
import jax.numpy as jnp
from jax import lax
from jax.experimental import pallas as pl
from jax.experimental.pallas import tpu as pltpu
from jax.experimental.pallas import tpu_sc as plsc

NC = 2
NS = 16
NW = NC * NS

EMB_DIM = 32
CHUNK = 128
K = 8
GROUP = K * CHUNK


def _body(idx_hbm, table_hbm, out_hbm, idx_v, rows0, rows1, g0, g1, s0, s1):
    nchunk = idx_hbm.shape[1]
    ngroups = nchunk // K
    wid = lax.axis_index("s") * NC + lax.axis_index("c")
    base = wid * (nchunk * CHUNK)
    pltpu.sync_copy(idx_hbm.at[wid], idx_v)

    bufs = (rows0, rows1)
    gsems = (g0, g1)
    ssems = (s0, s1)

    def fire_gathers(grp, b):
        for i in range(K):
            pltpu.async_copy(
                table_hbm.at[idx_v.at[grp * K + i]],
                bufs[b].at[pl.ds(i * CHUNK, CHUNK)],
                gsems[b])

    def drain_gathers(b):
        pltpu.make_async_copy(
            out_hbm.at[pl.ds(0, GROUP)], bufs[b], gsems[b]).wait()

    def store(grp, b):
        return pltpu.async_copy(
            bufs[b], out_hbm.at[pl.ds(base + grp * GROUP, GROUP)], ssems[b])

    fire_gathers(0, 0)
    fire_gathers(1, 1)

    @pl.loop(0, ngroups - 2, step=2)
    def _(g):
        drain_gathers(0)
        store(g, 0)
        drain_gathers(1)
        store(g + 1, 1)
        pltpu.make_async_copy(
            bufs[0], out_hbm.at[pl.ds(0, GROUP)], ssems[0]).wait()
        fire_gathers(g + 2, 0)
        pltpu.make_async_copy(
            bufs[1], out_hbm.at[pl.ds(0, GROUP)], ssems[1]).wait()
        fire_gathers(g + 3, 1)

    drain_gathers(0)
    store(ngroups - 2, 0).wait()
    drain_gathers(1)
    store(ngroups - 1, 1).wait()


def kernel(indices, emb_table):
    b, w = indices.shape
    n = b * w
    nchunk = n // (NW * CHUNK)
    idx3 = indices.astype(jnp.int32).reshape(NW, nchunk, CHUNK)

    run = pl.kernel(
        _body,
        out_type=jax.ShapeDtypeStruct((n, EMB_DIM), jnp.float32),
        mesh=plsc.VectorSubcoreMesh(
            core_axis_name="c", subcore_axis_name="s",
            num_cores=NC, num_subcores=NS),
        scratch_types=[
            pltpu.VMEM((nchunk, CHUNK), jnp.int32),
            pltpu.VMEM((GROUP, EMB_DIM), jnp.float32),
            pltpu.VMEM((GROUP, EMB_DIM), jnp.float32),
            pltpu.SemaphoreType.DMA,
            pltpu.SemaphoreType.DMA,
            pltpu.SemaphoreType.DMA,
            pltpu.SemaphoreType.DMA,
        ],
        compiler_params=pltpu.CompilerParams(use_tc_tiling_on_sc=False),
    )
    out = run(idx3, emb_table)
    return out.reshape(b, w, EMB_DIM)


# speedup vs baseline: 1.6021x; 1.0631x over previous
import jax
import jax.numpy as jnp
from jax import lax
from jax.experimental import pallas as pl
from jax.experimental.pallas import tpu as pltpu
from jax.experimental.pallas import tpu_sc as plsc

NC = 2
NS = 16
NW = NC * NS

EMB_DIM = 32
CHUNK = 128
K = 8
GROUP = K * CHUNK


def _body(idxT_hbm, table_hbm, out_hbm, idx_v, rows0, rows1, g0, g1, s0, s1):
    nwalk, nbatch = idxT_hbm.shape
    bp = nbatch // NW
    cpw = bp // CHUNK
    ngroups = (nwalk * cpw) // K
    t = lax.axis_index("s") * NC + lax.axis_index("c")
    b0 = t * bp

    pltpu.sync_copy(idxT_hbm.at[:, pl.ds(b0, bp)], idx_v)

    bufs = (rows0, rows1)
    gsems = (g0, g1)
    ssems = (s0, s1)
    wpg = K // cpw

    def fire_gathers(grp, b):
        for i in range(K):
            w = grp * wpg + i // cpw
            c = i % cpw
            pltpu.async_copy(
                table_hbm.at[idx_v.at[w, pl.ds(c * CHUNK, CHUNK)]],
                bufs[b].at[pl.ds(i * CHUNK, CHUNK)],
                gsems[b])

    def drain_gathers(b):
        pltpu.make_async_copy(
            out_hbm.at[0, pl.ds(0, GROUP)], bufs[b], gsems[b]).wait()

    def fire_stores(grp, b):
        for i in range(K):
            w = grp * wpg + i // cpw
            c = i % cpw
            pltpu.async_copy(
                bufs[b].at[pl.ds(i * CHUNK, CHUNK)],
                out_hbm.at[w, pl.ds(b0 + c * CHUNK, CHUNK)],
                ssems[b])

    def drain_stores(b):
        pltpu.make_async_copy(
            bufs[b], out_hbm.at[0, pl.ds(0, GROUP)], ssems[b]).wait()

    fire_gathers(0, 0)
    fire_gathers(1, 1)

    @pl.loop(0, ngroups - 2, step=2)
    def _(g):
        drain_gathers(0)
        fire_stores(g, 0)
        drain_gathers(1)
        fire_stores(g + 1, 1)
        drain_stores(0)
        fire_gathers(g + 2, 0)
        drain_stores(1)
        fire_gathers(g + 3, 1)

    drain_gathers(0)
    fire_stores(ngroups - 2, 0)
    drain_gathers(1)
    fire_stores(ngroups - 1, 1)
    drain_stores(0)
    drain_stores(1)


def kernel(indices, emb_table):
    b, w = indices.shape
    idxT = indices.astype(jnp.int32).T

    run = pl.kernel(
        _body,
        out_type=jax.ShapeDtypeStruct((w, b, EMB_DIM), jnp.float32),
        mesh=plsc.VectorSubcoreMesh(
            core_axis_name="c", subcore_axis_name="s",
            num_cores=NC, num_subcores=NS),
        scratch_types=[
            pltpu.VMEM((w, b // NW), jnp.int32),
            pltpu.VMEM((GROUP, EMB_DIM), jnp.float32),
            pltpu.VMEM((GROUP, EMB_DIM), jnp.float32),
            pltpu.SemaphoreType.DMA,
            pltpu.SemaphoreType.DMA,
            pltpu.SemaphoreType.DMA,
            pltpu.SemaphoreType.DMA,
        ],
        compiler_params=pltpu.CompilerParams(use_tc_tiling_on_sc=False),
    )
    out = run(idxT, emb_table)
    return out.transpose(1, 0, 2)
